# baseline (speedup 1.0000x reference)
import functools

import jax
import jax.numpy as jnp
from jax import lax
from jax.experimental import pallas as pl
from jax.experimental.pallas import tpu as pltpu
from jax.experimental.pallas import tpu_sc as plsc

_EMB = 64
_OUT = 2
_B = 4096
_L = 200
_NC = 2
_NS = 16
_NW = _NC * _NS
_SPW = _B // _NW
_G0 = 128
_G1 = _L - _G0
_LP = 256


def _pool_linear_body(text_hbm, table_hbm, w_hbm, b_hbm, out_hbm,
                      idx_v, rows_v, w_v, b_v, out_v, sem0, sem1):
    c = lax.axis_index("c")
    s = lax.axis_index("s")
    wid = s * _NC + c
    base = wid * _SPW
    pltpu.sync_copy(text_hbm.at[pl.ds(base, _SPW)], idx_v)
    pltpu.sync_copy(w_hbm, w_v)
    pltpu.sync_copy(b_hbm, b_v)

    w_rows = [[w_v[o, pl.ds(16 * k, 16)] for k in range(4)]
              for o in range(_OUT)]
    b_vec = b_v[...]
    lanes = lax.iota(jnp.int32, 16)
    sems = (sem0, sem1)
    inv_l = jnp.float32(1.0 / _L)

    def lane_sum(t):
        for sh in (8, 4, 2, 1):
            t = t + t.at[jnp.bitwise_xor(lanes, sh)].get(
                mode="promise_in_bounds")
        return t

    def fire(seq, buf):
        pltpu.make_async_copy(
            table_hbm.at[idx_v.at[seq, pl.ds(0, _G0)]],
            rows_v.at[buf, pl.ds(0, _G0)], sems[buf]).start()
        pltpu.make_async_copy(
            table_hbm.at[idx_v.at[seq, pl.ds(_G0, _G1)]],
            rows_v.at[buf, pl.ds(_G0, _G1)], sems[buf]).start()

    def process(seq, buf):
        pltpu.make_async_copy(
            table_hbm.at[pl.ds(0, _L)], rows_v.at[buf], sems[buf]).wait()

        def body(l, accs):
            return tuple(a + rows_v[buf, l, pl.ds(16 * k, 16)]
                         for k, a in enumerate(accs))

        accs = lax.fori_loop(
            0, _L, body,
            tuple(jnp.zeros((16,), jnp.float32) for _ in range(4)),
            unroll=10)
        outs = []
        for o in range(_OUT):
            t = accs[0] * w_rows[o][0]
            for k in range(1, 4):
                t = t + accs[k] * w_rows[o][k]
            outs.append(lane_sum(t))
        ov = jnp.where(lanes == 0, outs[0], outs[1]) * inv_l + b_vec
        out_v[seq] = ov

    fire(0, 0)

    def outer(i, carry):
        s0 = 2 * i
        fire(s0 + 1, 1)
        process(s0, 0)

        @pl.when(s0 + 2 < _SPW)
        def _():
            fire(s0 + 2, 0)

        process(s0 + 1, 1)
        return carry

    lax.fori_loop(0, _SPW // 2, outer, 0)
    pltpu.sync_copy(out_v, out_hbm.at[pl.ds(base, _SPW)])


_VOCAB = 1000000
_CH = 64
_NCHUNK = _VOCAB // _CH


def _flatten_body(table_hbm, out_hbm, in_v, flat_v, si0, si1, so0, so1):
    c = lax.axis_index("c")
    s = lax.axis_index("s")
    wid = s * _NC + c
    sin = (si0, si1)
    sout = (so0, so1)

    def fire_in(step, buf):
        cid = step * _NW + wid

        @pl.when(cid < _NCHUNK)
        def _():
            pltpu.make_async_copy(
                table_hbm.at[pl.ds(cid * _CH, _CH)],
                in_v.at[buf], sin[buf]).start()

    def compact(step, buf):
        cid = step * _NW + wid
        pltpu.make_async_copy(
            table_hbm.at[pl.ds(0, _CH)], in_v.at[buf], sin[buf]).wait()

        @pl.when(step >= 2)
        def _():
            pltpu.make_async_copy(
                flat_v.at[buf], out_hbm.at[pl.ds(0, _CH * _EMB)],
                sout[buf]).wait()

        def body(r, carry):
            for k in range(4):
                flat_v[buf, pl.ds(r * _EMB + 16 * k, 16)] = \
                    in_v[buf, r, pl.ds(16 * k, 16)]
            return carry

        lax.fori_loop(0, _CH, body, 0, unroll=4)
        pltpu.make_async_copy(
            flat_v.at[buf], out_hbm.at[pl.ds(cid * _CH * _EMB, _CH * _EMB)],
            sout[buf]).start()

    nstep = (_NCHUNK - wid + _NW - 1) // _NW
    fire_in(0, 0)

    def outer(i, carry):
        s0 = 2 * i

        @pl.when(s0 < nstep)
        def _():
            fire_in(s0 + 1, 1)
            compact(s0, 0)

        @pl.when(s0 + 1 < nstep)
        def _():
            fire_in(s0 + 2, 0)
            compact(s0 + 1, 1)

        return carry

    lax.fori_loop(0, (_NCHUNK // _NW + 2) // 2, outer, 0)
    for buf in range(2):
        pltpu.make_async_copy(
            flat_v.at[buf], out_hbm.at[pl.ds(0, _CH * _EMB)],
            sout[buf]).wait()


_flatten = functools.partial(
    pl.kernel,
    mesh=plsc.VectorSubcoreMesh(core_axis_name="c", subcore_axis_name="s"),
    out_type=jax.ShapeDtypeStruct((_VOCAB * _EMB,), jnp.float32),
    scratch_types=[
        pltpu.VMEM((2, _CH, _EMB), jnp.float32),
        pltpu.VMEM((2, _CH * _EMB), jnp.float32),
        pltpu.SemaphoreType.DMA,
        pltpu.SemaphoreType.DMA,
        pltpu.SemaphoreType.DMA,
        pltpu.SemaphoreType.DMA,
    ],
)(_flatten_body)


_pool_linear = functools.partial(
    pl.kernel,
    mesh=plsc.VectorSubcoreMesh(core_axis_name="c", subcore_axis_name="s"),
    out_type=jax.ShapeDtypeStruct((_B, 16), jnp.float32),
    compiler_params=pltpu.CompilerParams(use_tc_tiling_on_sc=False),
    scratch_types=[
        pltpu.VMEM((_SPW, _LP), jnp.int32),
        pltpu.VMEM((2, _L, _EMB), jnp.float32),
        pltpu.VMEM((_OUT, _EMB), jnp.float32),
        pltpu.VMEM((16,), jnp.float32),
        pltpu.VMEM((_SPW, 16), jnp.float32),
        pltpu.SemaphoreType.DMA,
        pltpu.SemaphoreType.DMA,
    ],
)(_pool_linear_body)


def kernel(text, emb_table, W, b):
    b16 = jnp.concatenate(
        [b.astype(jnp.float32), jnp.zeros((16 - _OUT,), jnp.float32)])
    text_p = jnp.pad(text.astype(jnp.int32), ((0, 0), (0, _LP - _L)))
    table_lin = _flatten(emb_table).reshape(emb_table.shape)
    return _pool_linear(text_p, table_lin, W, b16)[:, :_OUT]


# speedup vs baseline: 1.3901x; 1.3901x over previous
import functools

import jax
import jax.numpy as jnp
from jax import lax
from jax.experimental import pallas as pl
from jax.experimental.pallas import tpu as pltpu
from jax.experimental.pallas import tpu_sc as plsc

_EMB = 64
_OUT = 2
_B = 4096
_L = 200
_NC = 2
_NS = 16
_NW = _NC * _NS
_SPW = _B // _NW
_G0 = 128
_G1 = _L - _G0
_LP = 256


def _pool_linear_body(text_hbm, table_hbm, w_hbm, b_hbm, out_hbm,
                      idx_v, rows_v, w_v, b_v, out_v, sem0, sem1):
    c = lax.axis_index("c")
    s = lax.axis_index("s")
    wid = s * _NC + c
    base = wid * _SPW
    pltpu.sync_copy(text_hbm.at[pl.ds(base, _SPW)], idx_v)
    pltpu.sync_copy(w_hbm, w_v)
    pltpu.sync_copy(b_hbm, b_v)

    w_rows = [[w_v[o, pl.ds(16 * k, 16)] for k in range(4)]
              for o in range(_OUT)]
    b_vec = b_v[...]
    lanes = lax.iota(jnp.int32, 16)
    sems = (sem0, sem1)
    inv_l = jnp.float32(1.0 / _L)

    def lane_sum(t):
        for sh in (8, 4, 2, 1):
            t = t + t.at[jnp.bitwise_xor(lanes, sh)].get(
                mode="promise_in_bounds")
        return t

    def fire(seq, buf):
        pltpu.make_async_copy(
            table_hbm.at[idx_v.at[seq, pl.ds(0, _G0)]],
            rows_v.at[buf, pl.ds(0, _G0)], sems[buf]).start()
        pltpu.make_async_copy(
            table_hbm.at[idx_v.at[seq, pl.ds(_G0, _G1)]],
            rows_v.at[buf, pl.ds(_G0, _G1)], sems[buf]).start()

    def process(seq, buf):
        pltpu.make_async_copy(
            table_hbm.at[pl.ds(0, _L)], rows_v.at[buf], sems[buf]).wait()

        def body(l, accs):
            return tuple(a + rows_v[buf, l, pl.ds(16 * k, 16)]
                         for k, a in enumerate(accs))

        accs = lax.fori_loop(
            0, _L, body,
            tuple(jnp.zeros((16,), jnp.float32) for _ in range(4)),
            unroll=10)
        outs = []
        for o in range(_OUT):
            t = accs[0] * w_rows[o][0]
            for k in range(1, 4):
                t = t + accs[k] * w_rows[o][k]
            outs.append(lane_sum(t))
        ov = jnp.where(lanes == 0, outs[0], outs[1]) * inv_l + b_vec
        out_v[seq] = ov

    fire(0, 0)

    def outer(i, carry):
        s0 = 2 * i
        fire(s0 + 1, 1)
        process(s0, 0)

        @pl.when(s0 + 2 < _SPW)
        def _():
            fire(s0 + 2, 0)

        process(s0 + 1, 1)
        return carry

    lax.fori_loop(0, _SPW // 2, outer, 0)
    pltpu.sync_copy(out_v, out_hbm.at[pl.ds(base, _SPW)])


_pool_linear = functools.partial(
    pl.kernel,
    mesh=plsc.VectorSubcoreMesh(core_axis_name="c", subcore_axis_name="s"),
    out_type=jax.ShapeDtypeStruct((_B, 16), jnp.float32),
    compiler_params=pltpu.CompilerParams(use_tc_tiling_on_sc=False),
    scratch_types=[
        pltpu.VMEM((_SPW, _LP), jnp.int32),
        pltpu.VMEM((2, _L, _EMB), jnp.float32),
        pltpu.VMEM((_OUT, _EMB), jnp.float32),
        pltpu.VMEM((16,), jnp.float32),
        pltpu.VMEM((_SPW, 16), jnp.float32),
        pltpu.SemaphoreType.DMA,
        pltpu.SemaphoreType.DMA,
    ],
)(_pool_linear_body)


def kernel(text, emb_table, W, b):
    b16 = jnp.concatenate(
        [b.astype(jnp.float32), jnp.zeros((16 - _OUT,), jnp.float32)])
    text_p = jnp.pad(text.astype(jnp.int32), ((0, 0), (0, _LP - _L)))
    return _pool_linear(text_p, emb_table, W, b16)[:, :_OUT]
